# baseline (speedup 1.0000x reference)
import functools

import jax
import jax.numpy as jnp
from jax import lax
from jax.experimental import pallas as pl
from jax.experimental.pallas import tpu as pltpu
from jax.experimental.pallas import tpu_sc as plsc

BATCH = 16384
D = 8
NC, NS, L = 2, 16, 16
NW = NC * NS
BPW = BATCH // NW
CHUNK = 128
NCH = BPW // CHUNK
NG = BPW // L

_OFF_W1 = 0
_OFF_B1 = 136
_OFF_W2 = 144
_OFF_B2 = 176
_OFF_W3 = 180
_OFF_B3 = 184
_NPAR = 192


def _ncf_body(uidx_hbm, iidx_hbm, time_hbm, utab_hbm, itab_hbm, par_hbm,
              out_hbm,
              uidx_v, iidx_v, time_v, urows_v, irows_v, par_v, out_v,
              sem_u, sem_i):
    wid = lax.axis_index("s") * NC + lax.axis_index("c")
    base = wid * BPW

    pltpu.sync_copy(uidx_hbm.at[pl.ds(wid * NCH, NCH)], uidx_v)
    pltpu.sync_copy(iidx_hbm.at[pl.ds(wid * NCH, NCH)], iidx_v)

    copies = []
    for j in range(NCH):
        copies.append(pltpu.async_copy(
            utab_hbm.at[uidx_v.at[j]],
            urows_v.at[pl.ds(j * CHUNK, CHUNK)], sem_u))
        copies.append(pltpu.async_copy(
            itab_hbm.at[iidx_v.at[j]],
            irows_v.at[pl.ds(j * CHUNK, CHUNK)], sem_i))

    pltpu.sync_copy(time_hbm.at[pl.ds(base, BPW)], time_v)
    pltpu.sync_copy(par_hbm, par_v)

    for c in copies:
        c.wait()

    iota = lax.iota(jnp.int32, L)

    def group(g, carry):
        W1 = [[par_v[_OFF_W1 + j * 17 + k] for k in range(17)]
              for j in range(8)]
        b1 = [par_v[_OFF_B1 + j] for j in range(8)]
        W2 = [[par_v[_OFF_W2 + j * 8 + k] for k in range(8)]
              for j in range(4)]
        b2 = [par_v[_OFF_B2 + j] for j in range(4)]
        W3 = [par_v[_OFF_W3 + k] for k in range(4)]
        b3 = par_v[_OFF_B3]
        r0 = g * L
        rid = r0 + iota
        t = time_v[pl.ds(r0, L)]
        xu = [plsc.load_gather(urows_v, [rid, jnp.full((L,), d, jnp.int32)])
              for d in range(D)]
        xi = [plsc.load_gather(irows_v, [rid, jnp.full((L,), d, jnp.int32)])
              for d in range(D)]
        h1 = []
        for j in range(8):
            acc = t * W1[j][16] + b1[j]
            for k in range(8):
                acc = acc + xu[k] * W1[j][k]
            for k in range(8):
                acc = acc + xi[k] * W1[j][8 + k]
            h1.append(jnp.maximum(acc, 0.0))
        h2 = []
        for j in range(4):
            acc = h1[0] * W2[j][0] + b2[j]
            for k in range(1, 8):
                acc = acc + h1[k] * W2[j][k]
            h2.append(jnp.maximum(acc, 0.0))
        p = h2[0] * W3[0] + b3
        for k in range(1, 4):
            p = p + h2[k] * W3[k]
        out_v[pl.ds(r0, L)] = p
        return carry

    lax.fori_loop(0, NG, group, 0)
    pltpu.sync_copy(out_v, out_hbm.at[pl.ds(base, BPW)])


@functools.partial(jax.jit, static_argnums=())
def _ncf(uidx, iidx, time_input, user_table, item_table, par):
    f = pl.kernel(
        _ncf_body,
        out_type=jax.ShapeDtypeStruct((BATCH,), jnp.float32),
        mesh=plsc.VectorSubcoreMesh(core_axis_name="c", subcore_axis_name="s",
                                    num_cores=NC, num_subcores=NS),
        compiler_params=pltpu.CompilerParams(needs_layout_passes=False,
                                             use_tc_tiling_on_sc=False),
        scratch_types=[
            pltpu.VMEM((NCH, CHUNK), jnp.int32),
            pltpu.VMEM((NCH, CHUNK), jnp.int32),
            pltpu.VMEM((BPW,), jnp.float32),
            pltpu.VMEM((BPW, D), jnp.float32),
            pltpu.VMEM((BPW, D), jnp.float32),
            pltpu.VMEM((_NPAR, L), jnp.float32),
            pltpu.VMEM((BPW,), jnp.float32),
            pltpu.SemaphoreType.DMA,
            pltpu.SemaphoreType.DMA,
        ],
    )
    return f(uidx, iidx, time_input, user_table, item_table, par)


def kernel(user_input, item_input, time_input, user_table, item_table,
           W1, b1, W2, b2, W3, b3):
    par = jnp.concatenate([
        W1.reshape(-1), b1, W2.reshape(-1), b2, W3.reshape(-1), b3,
        jnp.zeros((_NPAR - 185,), jnp.float32)])
    par = jnp.tile(par[:, None], (1, L))
    uidx = user_input.reshape(NW * NCH, CHUNK)
    iidx = item_input.reshape(NW * NCH, CHUNK)
    pred = _ncf(uidx, iidx, time_input, user_table, item_table, par)
    return pred.reshape(BATCH, 1)
